# baseline (speedup 1.0000x reference)
import functools

import jax
import jax.numpy as jnp
from jax import lax
from jax.experimental import pallas as pl
from jax.experimental.pallas import tpu as pltpu
from jax.experimental.pallas import tpu_sc as plsc

N = 100000
E = 3200000
G = 128
IN, H1, H2, OUT = 5, 16, 32, 3

NC, NS, LANES = 2, 16, 16
NT = NC * NS
KS = 4
CI = 128
GROUP = KS * CI
GT = E // GROUP
ROWS = 102400
FR = ROWS // 8
RPT = ROWS // NS
FRPT = FR // NS
ZR = RPT // 16
PG = ROWS // (NT * CI)
NPT = PG * CI
PAD_BIN = G
GBINS = 136

_f32 = jnp.float32
_i32 = jnp.int32
_HIGH = lax.Precision.DEFAULT


def _vec_mesh():
  return plsc.VectorSubcoreMesh(core_axis_name="c", subcore_axis_name="s")


_SC_PARAMS = pltpu.CompilerParams(use_tc_tiling_on_sc=False)
_SC_PARAMS_NLP = pltpu.CompilerParams(use_tc_tiling_on_sc=False,
                                      needs_layout_passes=False)


_HR = ROWS // LANES
_HRT = _HR // NS
_CH = 8
_CHR = _HRT // _CH


def _deg_body(dst_hbm, out_hbm, deg_sp, hist, idxb, riex, buf, rep,
              si0, si1):
  c = lax.axis_index("c")
  s = lax.axis_index("s")
  q, r = divmod(GT, NT)
  w = c * NS + s
  base = w * q + jnp.minimum(w, r)
  ngt = q + jnp.where(w < r, 1, 0)
  nit = -(-(q + 1) // 2)
  si = (si0, si1)

  @pl.loop(0, _HR)
  def _(i):
    hist[i, :] = jnp.zeros((LANES,), _f32)

  @pl.loop(0, _CHR)
  def _(i):
    buf[i, :] = jnp.zeros((LANES,), _f32)

  for ch in range(_CH):
    pltpu.sync_copy(buf, deg_sp.at[pl.ds(s * _HRT + ch * _CHR, _CHR)])

  def fire_idx(g, b):
    pltpu.async_copy(dst_hbm.at[base + g], idxb.at[b], si[b])

  def wait_idx(g, b):
    pltpu.make_async_copy(dst_hbm.at[base + g], idxb.at[b], si[b]).wait()

  ones16 = jnp.ones((LANES,), _f32)
  fire_idx(0, 0)

  @pl.loop(0, nit)
  def _(g2):
    for b in (0, 1):
      g = g2 * 2 + b
      nb = 1 - b

      @pl.when(g < ngt)
      def _():
        wait_idx(g, b)

        @pl.when(g + 1 < ngt)
        def _():
          fire_idx(g + 1, nb)

        for j in range(KS):
          for kk in range(CI // LANES):
            iv = idxb[b, j, pl.ds(kk * LANES, LANES)]
            plsc.addupdate_scatter(
                hist, [iv >> 4, iv & 15], ones16)

  plsc.subcore_barrier()

  @pl.loop(0, _HR // CI)
  def _(t):
    @pl.loop(0, CI, step=LANES)
    def _(m):
      riex[pl.ds(m, LANES)] = (
          jnp.arange(LANES, dtype=_i32) + t * CI + m)

    pltpu.sync_copy(hist.at[pl.ds(t * CI, CI)], deg_sp.at[riex], add=True)

  plsc.subcore_barrier()

  for ch in range(_CH):
    pltpu.sync_copy(deg_sp.at[pl.ds(s * _HRT + ch * _CHR, _CHR)], buf)

    @pl.loop(0, _CHR)
    def _(i):
      v = buf[i, :]
      for k in range(16):
        rep[2 * i + k // 8, pl.ds((k % 8) * LANES, LANES)] = (
            jnp.full((LANES,), v[k], _f32))

    pltpu.sync_copy(
        rep, out_hbm.at[pl.ds(c * FR + s * FRPT + ch * (FRPT // _CH),
                              FRPT // _CH)])


def _run_deg(dst_r):
  return pl.kernel(
      _deg_body,
      out_type=jax.ShapeDtypeStruct((NC * FR, 128), _f32),
      mesh=_vec_mesh(),
      compiler_params=_SC_PARAMS_NLP,
      scratch_types=[
          pltpu.VMEM_SHARED((_HR, LANES), _f32),
          pltpu.VMEM((_HR, LANES), _f32),
          pltpu.VMEM((2, KS, CI), _i32),
          pltpu.VMEM((CI,), _i32),
          pltpu.VMEM((_CHR, LANES), _f32),
          pltpu.VMEM((FRPT // _CH, 128), _f32),
          pltpu.SemaphoreType.DMA,
          pltpu.SemaphoreType.DMA,
      ],
  )(dst_r)


def _agg_body(split_features, src_hbm, dst_hbm, tab_hbm, out_hbm,
              acc_sp, idxs, idxd, rows, zbuf,
              si0, si1, sg0, sg1, ss0, ss1):
  c = lax.axis_index("c")
  s = lax.axis_index("s")
  si = (si0, si1)
  sg = (sg0, sg1)
  ss = (ss0, ss1)

  @pl.loop(0, ZR)
  def _(i):
    zbuf[i, :] = jnp.zeros((LANES,), _f32)

  for k in range(16):
    pltpu.sync_copy(zbuf, acc_sp.at[pl.ds((s * 16 + k) * ZR, ZR)])
  plsc.subcore_barrier()

  tab = tab_hbm.at[c] if split_features else tab_hbm
  if split_features:
    q, r = divmod(GT, NS)
    w = s
  else:
    q, r = divmod(GT, NT)
    w = c * NS + s
  base = w * q + jnp.minimum(w, r)
  ngt = q + jnp.where(w < r, 1, 0)
  nit = -(-(q + 1) // 2)

  def fire_idx(g, b):
    pltpu.async_copy(src_hbm.at[base + g], idxs.at[b], si[b])
    pltpu.async_copy(dst_hbm.at[base + g], idxd.at[b], si[b])

  def wait_idx(g, b):
    pltpu.make_async_copy(src_hbm.at[base + g], idxs.at[b], si[b]).wait()
    pltpu.make_async_copy(dst_hbm.at[base + g], idxd.at[b], si[b]).wait()

  def fire_g(b):
    for j in range(KS):
      pltpu.async_copy(tab.at[idxs.at[b, j]], rows.at[b, j], sg[b])

  def wait_g(b):
    for j in range(KS):
      pltpu.make_async_copy(tab.at[idxs.at[b, j]], rows.at[b, j],
                            sg[b]).wait()

  def fire_s(b):
    for j in range(KS):
      pltpu.async_copy(rows.at[b, j], acc_sp.at[idxd.at[b, j]], ss[b],
                       add=True)

  def wait_s(b):
    for j in range(KS):
      pltpu.make_async_copy(rows.at[b, j], acc_sp.at[idxd.at[b, j]],
                            ss[b]).wait()

  fire_idx(0, 0)

  @pl.loop(0, nit)
  def _(g2):
    for b in (0, 1):
      g = g2 * 2 + b
      nb = 1 - b

      @pl.when(g < ngt)
      def _():
        wait_idx(g, b)
        fire_g(b)

        @pl.when(g > 0)
        def _():
          wait_s(nb)

        @pl.when(g + 1 < ngt)
        def _():
          fire_idx(g + 1, nb)

        wait_g(b)
        fire_s(b)

  for par in (0, 1):
    @pl.when((ngt - 1) % 2 == par)
    def _(par=par):
      wait_s(par)

  plsc.subcore_barrier()
  pltpu.sync_copy(acc_sp.at[pl.ds(s * RPT, RPT)],
                  out_hbm.at[c, pl.ds(s * RPT, RPT)])


def _run_agg(src_r, dst_r, table, split_features):
  return pl.kernel(
      functools.partial(_agg_body, split_features),
      out_type=jax.ShapeDtypeStruct((NC, ROWS, H1), _f32),
      mesh=_vec_mesh(),
      compiler_params=_SC_PARAMS,
      scratch_types=[
          pltpu.VMEM_SHARED((ROWS, H1), _f32),
          pltpu.VMEM((2, KS, CI), _i32),
          pltpu.VMEM((2, KS, CI), _i32),
          pltpu.VMEM((2, KS, CI, H1), _f32),
          pltpu.VMEM((ZR, H1), _f32),
          pltpu.SemaphoreType.DMA,
          pltpu.SemaphoreType.DMA,
          pltpu.SemaphoreType.DMA,
          pltpu.SemaphoreType.DMA,
          pltpu.SemaphoreType.DMA,
          pltpu.SemaphoreType.DMA,
      ],
  )(src_r, dst_r, table)


def _pool_body(h2_hbm, batch_hbm, s_out, c_out,
               sa_sp, sb_sp, cnt_sp, idxb, rows, ones_v, zb, zc,
               sl0, sl1, ss0, ss1):
  c = lax.axis_index("c")
  s = lax.axis_index("s")
  slab = c * NS + s
  sl = (sl0, sl1)
  ss = (ss0, ss1)

  @pl.when(s == 0)
  def _():
    @pl.loop(0, GBINS)
    def _(i):
      zb[i, :] = jnp.zeros((LANES,), _f32)

    @pl.loop(0, GBINS, step=LANES)
    def _(i):
      zc[pl.ds(i, LANES)] = jnp.zeros((LANES,), _f32)

    pltpu.sync_copy(zb, sa_sp)
    pltpu.sync_copy(zb, sb_sp)
    pltpu.sync_copy(zc, cnt_sp)

  @pl.loop(0, CI, step=LANES)
  def _(i):
    ones_v[pl.ds(i, LANES)] = jnp.ones((LANES,), _f32)

  plsc.subcore_barrier()
  pltpu.sync_copy(batch_hbm.at[slab], idxb)

  def fire_l(g, b):
    node0 = slab * NPT + g * CI
    pltpu.async_copy(h2_hbm.at[0, pl.ds(node0, CI)], rows.at[b, 0], sl[b])
    pltpu.async_copy(h2_hbm.at[1, pl.ds(node0, CI)], rows.at[b, 1], sl[b])

  def wait_l(g, b):
    node0 = slab * NPT + g * CI
    pltpu.make_async_copy(h2_hbm.at[0, pl.ds(node0, CI)], rows.at[b, 0],
                          sl[b]).wait()
    pltpu.make_async_copy(h2_hbm.at[1, pl.ds(node0, CI)], rows.at[b, 1],
                          sl[b]).wait()

  def fire_s(g, b):
    pltpu.async_copy(rows.at[b, 0], sa_sp.at[idxb.at[g]], ss[b], add=True)
    pltpu.async_copy(rows.at[b, 1], sb_sp.at[idxb.at[g]], ss[b], add=True)
    pltpu.async_copy(ones_v, cnt_sp.at[idxb.at[g]], ss[b], add=True)

  def wait_s(g, b):
    pltpu.make_async_copy(rows.at[b, 0], sa_sp.at[idxb.at[g]],
                          ss[b]).wait()
    pltpu.make_async_copy(rows.at[b, 1], sb_sp.at[idxb.at[g]],
                          ss[b]).wait()
    pltpu.make_async_copy(ones_v, cnt_sp.at[idxb.at[g]], ss[b]).wait()

  fire_l(0, 0)

  @pl.loop(0, (PG + 1) // 2)
  def _(g2):
    for b in (0, 1):
      g = g2 * 2 + b
      nb = 1 - b

      @pl.when(g < PG)
      def _():
        wait_l(g, b)

        @pl.when(g > 0)
        def _():
          wait_s(g - 1, nb)

        @pl.when(g + 1 < PG)
        def _():
          fire_l(g + 1, nb)

        fire_s(g, b)

  wait_s(PG - 1, (PG - 1) % 2)
  plsc.subcore_barrier()

  @pl.when(s == 0)
  def _():
    pltpu.sync_copy(sa_sp, zb)
    pltpu.sync_copy(zb, s_out.at[c, 0])
    pltpu.sync_copy(sb_sp, zb)
    pltpu.sync_copy(zb, s_out.at[c, 1])
    pltpu.sync_copy(cnt_sp, zc)
    pltpu.sync_copy(zc, c_out.at[pl.ds(c * GBINS, GBINS)])


def _run_pool(h2v, batch_r):
  return pl.kernel(
      _pool_body,
      out_type=(jax.ShapeDtypeStruct((NC, 2, GBINS, H1), _f32),
                jax.ShapeDtypeStruct((NC * GBINS,), _f32)),
      mesh=_vec_mesh(),
      compiler_params=_SC_PARAMS,
      scratch_types=[
          pltpu.VMEM_SHARED((GBINS, H1), _f32),
          pltpu.VMEM_SHARED((GBINS, H1), _f32),
          pltpu.VMEM_SHARED((GBINS,), _f32),
          pltpu.VMEM((PG, CI), _i32),
          pltpu.VMEM((2, 2, CI, H1), _f32),
          pltpu.VMEM((CI,), _f32),
          pltpu.VMEM((GBINS, H1), _f32),
          pltpu.VMEM((GBINS,), _f32),
          pltpu.SemaphoreType.DMA,
          pltpu.SemaphoreType.DMA,
          pltpu.SemaphoreType.DMA,
          pltpu.SemaphoreType.DMA,
      ],
  )(h2v, batch_r)


_FRB = 1600


def _tc_a_body(deg_ref, xf_ref, w1f_ref, ht1_ref, di_ref):
  di = lax.rsqrt(deg_ref[0] + deg_ref[1] + 1.0)
  h = jnp.dot(xf_ref[...], w1f_ref[...], precision=_HIGH,
              preferred_element_type=_f32)
  ht1_ref[...] = h * di
  di_ref[...] = di


def _tc_a(deg16, xf, W1f):
  return pl.pallas_call(
      _tc_a_body,
      grid=(FR // _FRB,),
      in_specs=[
          pl.BlockSpec((NC, _FRB, 128), lambda i: (0, i, 0)),
          pl.BlockSpec((_FRB, 64), lambda i: (i, 0)),
          pl.BlockSpec((64, 128), lambda i: (0, 0)),
      ],
      out_specs=[
          pl.BlockSpec((_FRB, 128), lambda i: (i, 0)),
          pl.BlockSpec((_FRB, 128), lambda i: (i, 0)),
      ],
      out_shape=[
          jax.ShapeDtypeStruct((FR, 128), _f32),
          jax.ShapeDtypeStruct((FR, 128), _f32),
      ],
  )(deg16, xf, W1f)


def _tc_b_body(acc_ref, ht1_ref, di_ref, b1f_ref, w2f_ref, ht2_ref):
  di = di_ref[...]
  h1 = jnp.maximum((acc_ref[0] + acc_ref[1] + ht1_ref[...]) * di
                   + b1f_ref[...], 0.0)
  ht2 = jnp.dot(h1, w2f_ref[...], precision=_HIGH,
                preferred_element_type=_f32)
  ht2_ref[0] = ht2[:, :128] * di
  ht2_ref[1] = ht2[:, 128:] * di


def _tc_b(acc1f, ht1f, di16, b1f, W2f):
  return pl.pallas_call(
      _tc_b_body,
      grid=(FR // _FRB,),
      in_specs=[
          pl.BlockSpec((NC, _FRB, 128), lambda i: (0, i, 0)),
          pl.BlockSpec((_FRB, 128), lambda i: (i, 0)),
          pl.BlockSpec((_FRB, 128), lambda i: (i, 0)),
          pl.BlockSpec((1, 128), lambda i: (0, 0)),
          pl.BlockSpec((128, 256), lambda i: (0, 0)),
      ],
      out_specs=[pl.BlockSpec((NC, _FRB, 128), lambda i: (0, i, 0))],
      out_shape=[jax.ShapeDtypeStruct((NC, FR, 128), _f32)],
  )(acc1f, ht1f, di16, b1f, W2f)[0]


def _tc_c_body(acc_ref, ht2_ref, di_ref, b2f_ref, h2_ref):
  di = di_ref[...]
  h2_ref[0] = jnp.maximum((acc_ref[0] + ht2_ref[0]) * di + b2f_ref[0], 0.0)
  h2_ref[1] = jnp.maximum((acc_ref[1] + ht2_ref[1]) * di + b2f_ref[1], 0.0)


def _tc_c(acc2f, ht2f, di16, b2f):
  return pl.pallas_call(
      _tc_c_body,
      grid=(FR // _FRB,),
      in_specs=[
          pl.BlockSpec((NC, _FRB, 128), lambda i: (0, i, 0)),
          pl.BlockSpec((NC, _FRB, 128), lambda i: (0, i, 0)),
          pl.BlockSpec((_FRB, 128), lambda i: (i, 0)),
          pl.BlockSpec((NC, 1, 128), lambda i: (0, 0, 0)),
      ],
      out_specs=[pl.BlockSpec((NC, _FRB, 128), lambda i: (0, i, 0))],
      out_shape=[jax.ShapeDtypeStruct((NC, FR, 128), _f32)],
  )(acc2f, ht2f, di16, b2f)[0]


def _tc_d_body(s4_ref, cntT_ref, wf_ref, bf_ref, out_ref):
  sa = s4_ref[0, 0] + s4_ref[1, 0]
  sb = s4_ref[0, 1] + s4_ref[1, 1]
  ssum = jnp.concatenate([sa, sb], axis=1)
  cnt = cntT_ref[:, 0:1] + cntT_ref[:, 1:2]
  pooled = ssum / jnp.maximum(cnt, 1.0)
  o = jnp.dot(pooled, wf_ref[...], precision=_HIGH,
              preferred_element_type=_f32) + bf_ref[...]
  out_ref[...] = o[:G]


def _tc_d(s4, cntT, Wf, bf):
  return pl.pallas_call(
      _tc_d_body,
      grid=(1,),
      in_specs=[
          pl.BlockSpec((NC, 2, GBINS, H1), lambda i: (0, 0, 0, 0)),
          pl.BlockSpec((GBINS, NC), lambda i: (0, 0)),
          pl.BlockSpec((H2, OUT), lambda i: (0, 0)),
          pl.BlockSpec((1, OUT), lambda i: (0, 0)),
      ],
      out_specs=[pl.BlockSpec((G, OUT), lambda i: (0, 0))],
      out_shape=[jax.ShapeDtypeStruct((G, OUT), _f32)],
  )(s4, cntT, Wf, bf)[0]


def kernel(x, edge_index, batch, W1, b1, W2, b2, Wf, bf):
  src_r = edge_index[0].reshape(GT, KS, CI)
  dst_r = edge_index[1].reshape(GT, KS, CI)
  batch_r = jnp.concatenate(
      [batch, jnp.full((ROWS - N,), PAD_BIN, _i32)]).reshape(NT, PG, CI)
  xf = jnp.pad(x, ((0, ROWS - N), (0, 8 - IN))).reshape(FR, 64)

  eye8 = jnp.eye(8, dtype=_f32)
  W1f = jnp.kron(eye8, jnp.pad(W1, ((0, 8 - IN), (0, 0))))
  W2f = jnp.concatenate(
      [jnp.kron(eye8, W2[:, :H1]), jnp.kron(eye8, W2[:, H1:])],
      axis=1)
  b1f = jnp.tile(b1, 8).reshape(1, 128)
  b2f = jnp.stack([jnp.tile(b2[:H1], 8), jnp.tile(b2[H1:], 8)]
                  ).reshape(NC, 1, 128)

  deg16 = _run_deg(dst_r).reshape(NC, FR, 128)
  ht1f, di16 = _tc_a(deg16, xf, W1f)
  acc1 = _run_agg(src_r, dst_r, ht1f.reshape(ROWS, H1),
                  split_features=False)
  ht2f = _tc_b(acc1.reshape(NC, FR, 128), ht1f, di16, b1f, W2f)
  acc2 = _run_agg(src_r, dst_r, ht2f.reshape(NC, ROWS, H1),
                  split_features=True)
  h2f = _tc_c(acc2.reshape(NC, FR, 128), ht2f, di16, b2f)
  s4, cnt2 = _run_pool(h2f.reshape(NC, ROWS, H1), batch_r)
  return _tc_d(s4, cnt2.reshape(NC, GBINS).T, Wf, bf.reshape(1, OUT))


# speedup vs baseline: 1.0215x; 1.0215x over previous
import functools

import jax
import jax.numpy as jnp
from jax import lax
from jax.experimental import pallas as pl
from jax.experimental.pallas import tpu as pltpu
from jax.experimental.pallas import tpu_sc as plsc

N = 100000
E = 3200000
G = 128
IN, H1, H2, OUT = 5, 16, 32, 3

NC, NS, LANES = 2, 16, 16
NT = NC * NS
KS = 4
CI = 128
GROUP = KS * CI
GT = E // GROUP
ROWS = 102400
FR = ROWS // 8
RPT = ROWS // NS
FRPT = FR // NS
ZR = RPT // 16
PG = ROWS // (NT * CI)
NPT = PG * CI
PAD_BIN = G
GBINS = 136

_f32 = jnp.float32
_i32 = jnp.int32
_HIGH = lax.Precision.DEFAULT


def _vec_mesh():
  return plsc.VectorSubcoreMesh(core_axis_name="c", subcore_axis_name="s")


_SC_PARAMS = pltpu.CompilerParams(use_tc_tiling_on_sc=False)
_SC_PARAMS_NLP = pltpu.CompilerParams(use_tc_tiling_on_sc=False,
                                      needs_layout_passes=False)


def _deg_body(dst_hbm, out_hbm, deg_sp, idxb, ones_v, degb, rep,
              si0, si1, ss0, ss1):
  c = lax.axis_index("c")
  s = lax.axis_index("s")
  q, r = divmod(GT, NT)
  w = c * NS + s
  base = w * q + jnp.minimum(w, r)
  ngt = q + jnp.where(w < r, 1, 0)
  nit = -(-(q + 1) // 2)
  si = (si0, si1)
  ss = (ss0, ss1)

  @pl.loop(0, RPT, step=LANES)
  def _(i):
    degb[pl.ds(i, LANES)] = jnp.zeros((LANES,), _f32)

  @pl.loop(0, CI, step=LANES)
  def _(i):
    ones_v[pl.ds(i, LANES)] = jnp.ones((LANES,), _f32)

  pltpu.sync_copy(degb, deg_sp.at[pl.ds(s * RPT, RPT)])
  plsc.subcore_barrier()

  def fire_idx(g, b):
    pltpu.async_copy(dst_hbm.at[base + g], idxb.at[b], si[b])

  def wait_idx(g, b):
    pltpu.make_async_copy(dst_hbm.at[base + g], idxb.at[b], si[b]).wait()

  def fire_s(b):
    for j in range(KS):
      pltpu.async_copy(ones_v, deg_sp.at[idxb.at[b, j]], ss[b], add=True)

  def wait_s(b):
    for j in range(KS):
      pltpu.make_async_copy(ones_v, deg_sp.at[idxb.at[b, j]], ss[b]).wait()

  fire_idx(0, 0)

  @pl.loop(0, nit)
  def _(g2):
    for b in (0, 1):
      g = g2 * 2 + b
      nb = 1 - b

      @pl.when(g < ngt)
      def _():
        wait_idx(g, b)

        @pl.when(g > 0)
        def _():
          wait_s(nb)

        @pl.when(g + 1 < ngt)
        def _():
          fire_idx(g + 1, nb)

        fire_s(b)

  for par in (0, 1):
    @pl.when((ngt - 1) % 2 == par)
    def _(par=par):
      wait_s(par)

  plsc.subcore_barrier()
  pltpu.sync_copy(deg_sp.at[pl.ds(s * RPT, RPT)], degb)

  @pl.loop(0, FRPT // 2)
  def _(rr):
    v = degb[pl.ds(rr * 2 * 8, LANES)]
    for k in range(16):
      rep[rr * 2 + k // 8, pl.ds((k % 8) * LANES, LANES)] = (
          jnp.full((LANES,), v[k], _f32))

  pltpu.sync_copy(rep, out_hbm.at[pl.ds(c * FR + s * FRPT, FRPT)])


def _run_deg(dst_r):
  return pl.kernel(
      _deg_body,
      out_type=jax.ShapeDtypeStruct((NC * FR, 128), _f32),
      mesh=_vec_mesh(),
      compiler_params=_SC_PARAMS,
      scratch_types=[
          pltpu.VMEM_SHARED((ROWS,), _f32),
          pltpu.VMEM((2, KS, CI), _i32),
          pltpu.VMEM((CI,), _f32),
          pltpu.VMEM((RPT,), _f32),
          pltpu.VMEM((FRPT, 128), _f32),
          pltpu.SemaphoreType.DMA,
          pltpu.SemaphoreType.DMA,
          pltpu.SemaphoreType.DMA,
          pltpu.SemaphoreType.DMA,
      ],
  )(dst_r)


def _agg_body(split_features, src_hbm, dst_hbm, tab_hbm, out_hbm,
              acc_sp, idxs, idxd, rows, zbuf,
              si0, si1, sg0, sg1, ss0, ss1):
  c = lax.axis_index("c")
  s = lax.axis_index("s")
  si = (si0, si1)
  sg = (sg0, sg1)
  ss = (ss0, ss1)

  @pl.loop(0, ZR)
  def _(i):
    zbuf[i, :] = jnp.zeros((LANES,), _f32)

  for k in range(16):
    pltpu.sync_copy(zbuf, acc_sp.at[pl.ds((s * 16 + k) * ZR, ZR)])
  plsc.subcore_barrier()

  tab = tab_hbm.at[c] if split_features else tab_hbm
  if split_features:
    q, r = divmod(GT, NS)
    w = s
  else:
    q, r = divmod(GT, NT)
    w = c * NS + s
  base = w * q + jnp.minimum(w, r)
  ngt = q + jnp.where(w < r, 1, 0)
  nit = -(-(q + 1) // 2)

  def fire_idx(g, b):
    pltpu.async_copy(src_hbm.at[base + g], idxs.at[b], si[b])
    pltpu.async_copy(dst_hbm.at[base + g], idxd.at[b], si[b])

  def wait_idx(g, b):
    pltpu.make_async_copy(src_hbm.at[base + g], idxs.at[b], si[b]).wait()
    pltpu.make_async_copy(dst_hbm.at[base + g], idxd.at[b], si[b]).wait()

  def fire_g(b):
    for j in range(KS):
      pltpu.async_copy(tab.at[idxs.at[b, j]], rows.at[b, j], sg[b])

  def wait_g(b):
    for j in range(KS):
      pltpu.make_async_copy(tab.at[idxs.at[b, j]], rows.at[b, j],
                            sg[b]).wait()

  def fire_s(b):
    for j in range(KS):
      pltpu.async_copy(rows.at[b, j], acc_sp.at[idxd.at[b, j]], ss[b],
                       add=True)

  def wait_s(b):
    for j in range(KS):
      pltpu.make_async_copy(rows.at[b, j], acc_sp.at[idxd.at[b, j]],
                            ss[b]).wait()

  fire_idx(0, 0)

  @pl.loop(0, nit)
  def _(g2):
    for b in (0, 1):
      g = g2 * 2 + b
      nb = 1 - b

      @pl.when(g < ngt)
      def _():
        wait_idx(g, b)
        fire_g(b)

        @pl.when(g > 0)
        def _():
          wait_s(nb)

        @pl.when(g + 1 < ngt)
        def _():
          fire_idx(g + 1, nb)

        wait_g(b)
        fire_s(b)

  for par in (0, 1):
    @pl.when((ngt - 1) % 2 == par)
    def _(par=par):
      wait_s(par)

  plsc.subcore_barrier()
  pltpu.sync_copy(acc_sp.at[pl.ds(s * RPT, RPT)],
                  out_hbm.at[c, pl.ds(s * RPT, RPT)])


def _run_agg(src_r, dst_r, table, split_features):
  return pl.kernel(
      functools.partial(_agg_body, split_features),
      out_type=jax.ShapeDtypeStruct((NC, ROWS, H1), _f32),
      mesh=_vec_mesh(),
      compiler_params=_SC_PARAMS,
      scratch_types=[
          pltpu.VMEM_SHARED((ROWS, H1), _f32),
          pltpu.VMEM((2, KS, CI), _i32),
          pltpu.VMEM((2, KS, CI), _i32),
          pltpu.VMEM((2, KS, CI, H1), _f32),
          pltpu.VMEM((ZR, H1), _f32),
          pltpu.SemaphoreType.DMA,
          pltpu.SemaphoreType.DMA,
          pltpu.SemaphoreType.DMA,
          pltpu.SemaphoreType.DMA,
          pltpu.SemaphoreType.DMA,
          pltpu.SemaphoreType.DMA,
      ],
  )(src_r, dst_r, table)


def _pool_body(h2_hbm, batch_hbm, s_out, c_out,
               sa_sp, sb_sp, cnt_sp, idxb, rows, ones_v, zb, zc,
               sl0, sl1, ss0, ss1):
  c = lax.axis_index("c")
  s = lax.axis_index("s")
  slab = c * NS + s
  sl = (sl0, sl1)
  ss = (ss0, ss1)

  @pl.when(s == 0)
  def _():
    @pl.loop(0, GBINS)
    def _(i):
      zb[i, :] = jnp.zeros((LANES,), _f32)

    @pl.loop(0, GBINS, step=LANES)
    def _(i):
      zc[pl.ds(i, LANES)] = jnp.zeros((LANES,), _f32)

    pltpu.sync_copy(zb, sa_sp)
    pltpu.sync_copy(zb, sb_sp)
    pltpu.sync_copy(zc, cnt_sp)

  @pl.loop(0, CI, step=LANES)
  def _(i):
    ones_v[pl.ds(i, LANES)] = jnp.ones((LANES,), _f32)

  plsc.subcore_barrier()
  pltpu.sync_copy(batch_hbm.at[slab], idxb)

  def fire_l(g, b):
    node0 = slab * NPT + g * CI
    pltpu.async_copy(h2_hbm.at[0, pl.ds(node0, CI)], rows.at[b, 0], sl[b])
    pltpu.async_copy(h2_hbm.at[1, pl.ds(node0, CI)], rows.at[b, 1], sl[b])

  def wait_l(g, b):
    node0 = slab * NPT + g * CI
    pltpu.make_async_copy(h2_hbm.at[0, pl.ds(node0, CI)], rows.at[b, 0],
                          sl[b]).wait()
    pltpu.make_async_copy(h2_hbm.at[1, pl.ds(node0, CI)], rows.at[b, 1],
                          sl[b]).wait()

  def fire_s(g, b):
    pltpu.async_copy(rows.at[b, 0], sa_sp.at[idxb.at[g]], ss[b], add=True)
    pltpu.async_copy(rows.at[b, 1], sb_sp.at[idxb.at[g]], ss[b], add=True)
    pltpu.async_copy(ones_v, cnt_sp.at[idxb.at[g]], ss[b], add=True)

  def wait_s(g, b):
    pltpu.make_async_copy(rows.at[b, 0], sa_sp.at[idxb.at[g]],
                          ss[b]).wait()
    pltpu.make_async_copy(rows.at[b, 1], sb_sp.at[idxb.at[g]],
                          ss[b]).wait()
    pltpu.make_async_copy(ones_v, cnt_sp.at[idxb.at[g]], ss[b]).wait()

  fire_l(0, 0)

  @pl.loop(0, (PG + 1) // 2)
  def _(g2):
    for b in (0, 1):
      g = g2 * 2 + b
      nb = 1 - b

      @pl.when(g < PG)
      def _():
        wait_l(g, b)

        @pl.when(g > 0)
        def _():
          wait_s(g - 1, nb)

        @pl.when(g + 1 < PG)
        def _():
          fire_l(g + 1, nb)

        fire_s(g, b)

  wait_s(PG - 1, (PG - 1) % 2)
  plsc.subcore_barrier()

  @pl.when(s == 0)
  def _():
    pltpu.sync_copy(sa_sp, zb)
    pltpu.sync_copy(zb, s_out.at[c, 0])
    pltpu.sync_copy(sb_sp, zb)
    pltpu.sync_copy(zb, s_out.at[c, 1])
    pltpu.sync_copy(cnt_sp, zc)
    pltpu.sync_copy(zc, c_out.at[pl.ds(c * GBINS, GBINS)])


def _run_pool(h2v, batch_r):
  return pl.kernel(
      _pool_body,
      out_type=(jax.ShapeDtypeStruct((NC, 2, GBINS, H1), _f32),
                jax.ShapeDtypeStruct((NC * GBINS,), _f32)),
      mesh=_vec_mesh(),
      compiler_params=_SC_PARAMS,
      scratch_types=[
          pltpu.VMEM_SHARED((GBINS, H1), _f32),
          pltpu.VMEM_SHARED((GBINS, H1), _f32),
          pltpu.VMEM_SHARED((GBINS,), _f32),
          pltpu.VMEM((PG, CI), _i32),
          pltpu.VMEM((2, 2, CI, H1), _f32),
          pltpu.VMEM((CI,), _f32),
          pltpu.VMEM((GBINS, H1), _f32),
          pltpu.VMEM((GBINS,), _f32),
          pltpu.SemaphoreType.DMA,
          pltpu.SemaphoreType.DMA,
          pltpu.SemaphoreType.DMA,
          pltpu.SemaphoreType.DMA,
      ],
  )(h2v, batch_r)


_FRB = 1600


def _tc_a_body(deg_ref, xf_ref, w1f_ref, ht1_ref, di_ref):
  di = lax.rsqrt(deg_ref[0] + deg_ref[1] + 1.0)
  h = jnp.dot(xf_ref[...], w1f_ref[...], precision=_HIGH,
              preferred_element_type=_f32)
  ht1_ref[...] = h * di
  di_ref[...] = di


def _tc_a(deg16, xf, W1f):
  return pl.pallas_call(
      _tc_a_body,
      grid=(FR // _FRB,),
      in_specs=[
          pl.BlockSpec((NC, _FRB, 128), lambda i: (0, i, 0)),
          pl.BlockSpec((_FRB, 64), lambda i: (i, 0)),
          pl.BlockSpec((64, 128), lambda i: (0, 0)),
      ],
      out_specs=[
          pl.BlockSpec((_FRB, 128), lambda i: (i, 0)),
          pl.BlockSpec((_FRB, 128), lambda i: (i, 0)),
      ],
      out_shape=[
          jax.ShapeDtypeStruct((FR, 128), _f32),
          jax.ShapeDtypeStruct((FR, 128), _f32),
      ],
  )(deg16, xf, W1f)


def _tc_b_body(acc_ref, ht1_ref, di_ref, b1f_ref, w2f_ref, ht2_ref):
  di = di_ref[...]
  h1 = jnp.maximum((acc_ref[0] + acc_ref[1] + ht1_ref[...]) * di
                   + b1f_ref[...], 0.0)
  ht2 = jnp.dot(h1, w2f_ref[...], precision=_HIGH,
                preferred_element_type=_f32)
  ht2_ref[0] = ht2[:, :128] * di
  ht2_ref[1] = ht2[:, 128:] * di


def _tc_b(acc1f, ht1f, di16, b1f, W2f):
  return pl.pallas_call(
      _tc_b_body,
      grid=(FR // _FRB,),
      in_specs=[
          pl.BlockSpec((NC, _FRB, 128), lambda i: (0, i, 0)),
          pl.BlockSpec((_FRB, 128), lambda i: (i, 0)),
          pl.BlockSpec((_FRB, 128), lambda i: (i, 0)),
          pl.BlockSpec((1, 128), lambda i: (0, 0)),
          pl.BlockSpec((128, 256), lambda i: (0, 0)),
      ],
      out_specs=[pl.BlockSpec((NC, _FRB, 128), lambda i: (0, i, 0))],
      out_shape=[jax.ShapeDtypeStruct((NC, FR, 128), _f32)],
  )(acc1f, ht1f, di16, b1f, W2f)[0]


def _tc_c_body(acc_ref, ht2_ref, di_ref, b2f_ref, h2_ref):
  di = di_ref[...]
  h2_ref[0] = jnp.maximum((acc_ref[0] + ht2_ref[0]) * di + b2f_ref[0], 0.0)
  h2_ref[1] = jnp.maximum((acc_ref[1] + ht2_ref[1]) * di + b2f_ref[1], 0.0)


def _tc_c(acc2f, ht2f, di16, b2f):
  return pl.pallas_call(
      _tc_c_body,
      grid=(FR // _FRB,),
      in_specs=[
          pl.BlockSpec((NC, _FRB, 128), lambda i: (0, i, 0)),
          pl.BlockSpec((NC, _FRB, 128), lambda i: (0, i, 0)),
          pl.BlockSpec((_FRB, 128), lambda i: (i, 0)),
          pl.BlockSpec((NC, 1, 128), lambda i: (0, 0, 0)),
      ],
      out_specs=[pl.BlockSpec((NC, _FRB, 128), lambda i: (0, i, 0))],
      out_shape=[jax.ShapeDtypeStruct((NC, FR, 128), _f32)],
  )(acc2f, ht2f, di16, b2f)[0]


def _tc_d_body(s4_ref, cntT_ref, wf_ref, bf_ref, out_ref):
  sa = s4_ref[0, 0] + s4_ref[1, 0]
  sb = s4_ref[0, 1] + s4_ref[1, 1]
  ssum = jnp.concatenate([sa, sb], axis=1)
  cnt = cntT_ref[:, 0:1] + cntT_ref[:, 1:2]
  pooled = ssum / jnp.maximum(cnt, 1.0)
  o = jnp.dot(pooled, wf_ref[...], precision=_HIGH,
              preferred_element_type=_f32) + bf_ref[...]
  out_ref[...] = o[:G]


def _tc_d(s4, cntT, Wf, bf):
  return pl.pallas_call(
      _tc_d_body,
      grid=(1,),
      in_specs=[
          pl.BlockSpec((NC, 2, GBINS, H1), lambda i: (0, 0, 0, 0)),
          pl.BlockSpec((GBINS, NC), lambda i: (0, 0)),
          pl.BlockSpec((H2, OUT), lambda i: (0, 0)),
          pl.BlockSpec((1, OUT), lambda i: (0, 0)),
      ],
      out_specs=[pl.BlockSpec((G, OUT), lambda i: (0, 0))],
      out_shape=[jax.ShapeDtypeStruct((G, OUT), _f32)],
  )(s4, cntT, Wf, bf)[0]


def kernel(x, edge_index, batch, W1, b1, W2, b2, Wf, bf):
  src_r = edge_index[0].reshape(GT, KS, CI)
  dst_r = edge_index[1].reshape(GT, KS, CI)
  batch_r = jnp.concatenate(
      [batch, jnp.full((ROWS - N,), PAD_BIN, _i32)]).reshape(NT, PG, CI)
  xf = jnp.pad(x, ((0, ROWS - N), (0, 8 - IN))).reshape(FR, 64)

  eye8 = jnp.eye(8, dtype=_f32)
  W1f = jnp.kron(eye8, jnp.pad(W1, ((0, 8 - IN), (0, 0))))
  W2f = jnp.concatenate(
      [jnp.kron(eye8, W2[:, :H1]), jnp.kron(eye8, W2[:, H1:])],
      axis=1)
  b1f = jnp.tile(b1, 8).reshape(1, 128)
  b2f = jnp.stack([jnp.tile(b2[:H1], 8), jnp.tile(b2[H1:], 8)]
                  ).reshape(NC, 1, 128)

  deg16 = _run_deg(dst_r).reshape(NC, FR, 128)
  ht1f, di16 = _tc_a(deg16, xf, W1f)
  acc1 = _run_agg(src_r, dst_r, ht1f.reshape(ROWS, H1),
                  split_features=False)
  ht2f = _tc_b(acc1.reshape(NC, FR, 128), ht1f, di16, b1f, W2f)
  acc2 = _run_agg(src_r, dst_r, ht2f.reshape(NC, ROWS, H1),
                  split_features=True)
  h2f = _tc_c(acc2.reshape(NC, FR, 128), ht2f, di16, b2f)
  s4, cnt2 = _run_pool(h2f.reshape(NC, ROWS, H1), batch_r)
  return _tc_d(s4, cnt2.reshape(NC, GBINS).T, Wf, bf.reshape(1, OUT))
